# initial kernel scaffold (unmeasured)
import jax
import jax.numpy as jnp
from jax import lax
from jax.experimental import pallas as pl
from jax.experimental.pallas import tpu as pltpu

N_DEV = 8
N_STEPS = N_DEV - 1
N_SLOTS = 4


def kernel(x, w_mat):
    m_per, k = x.shape
    _, n_per = w_mat.shape
    half = m_per // 2

    def body(x_ref, w_ref, out_ref, cw_buf, ccw_buf,
             cw_send_sems, cw_recv_sems, ccw_send_sems, ccw_recv_sems,
             cw_credit, ccw_credit):
        my = lax.axis_index("i")
        right = jnp.mod(my + 1, N_DEV)
        left = jnp.mod(my - 1, N_DEV)

        barrier = pltpu.get_barrier_semaphore()
        pl.semaphore_signal(barrier, inc=1, device_id=(left,),
                            device_id_type=pl.DeviceIdType.MESH)
        pl.semaphore_signal(barrier, inc=1, device_id=(right,),
                            device_id_type=pl.DeviceIdType.MESH)
        pl.semaphore_wait(barrier, 2)

        def desc(s, buf, send_sems, recv_sems, tgt, row0):
            slot = s % N_SLOTS
            if s == 0:
                src = x_ref.at[pl.ds(row0, half), :]
            else:
                src = buf.at[(s - 1) % N_SLOTS]
            return pltpu.make_async_remote_copy(
                src_ref=src,
                dst_ref=buf.at[slot],
                send_sem=send_sems.at[slot],
                recv_sem=recv_sems.at[slot],
                device_id=(tgt,),
                device_id_type=pl.DeviceIdType.MESH,
            )

        def cw_desc(s):
            return desc(s, cw_buf, cw_send_sems, cw_recv_sems, right, 0)

        def ccw_desc(s):
            return desc(s, ccw_buf, ccw_send_sems, ccw_recv_sems, left, half)

        cw_desc(0).start()
        ccw_desc(0).start()

        acc = jnp.dot(x_ref[...], w_ref[...],
                      preferred_element_type=jnp.float32)
        out_ref[pl.ds(my * m_per, m_per), :] = jnp.maximum(acc, 0.0)

        for h in range(N_STEPS):
            cw_desc(h).wait_recv()
            if h + 1 < N_STEPS:
                if h + 1 >= N_SLOTS:
                    pl.semaphore_wait(cw_credit, 1)
                cw_desc(h + 1).start()
            ccw_desc(h).wait_recv()
            if h + 1 < N_STEPS:
                if h + 1 >= N_SLOTS:
                    pl.semaphore_wait(ccw_credit, 1)
                ccw_desc(h + 1).start()

            cw_desc(h).wait_send()
            ccw_desc(h).wait_send()

            if 1 <= h <= N_STEPS - N_SLOTS:
                pl.semaphore_signal(cw_credit, inc=1, device_id=(left,),
                                    device_id_type=pl.DeviceIdType.MESH)
                pl.semaphore_signal(ccw_credit, inc=1, device_id=(right,),
                                    device_id_type=pl.DeviceIdType.MESH)

            o_cw = jnp.mod(my - 1 - h, N_DEV)
            y = jnp.dot(cw_buf[h % N_SLOTS], w_ref[...],
                        preferred_element_type=jnp.float32)
            out_ref[pl.ds(o_cw * m_per, half), :] = jnp.maximum(y, 0.0)

            o_ccw = jnp.mod(my + 1 + h, N_DEV)
            y = jnp.dot(ccw_buf[h % N_SLOTS], w_ref[...],
                        preferred_element_type=jnp.float32)
            out_ref[pl.ds(o_ccw * m_per + half, half), :] = jnp.maximum(y, 0.0)

    out_shape = jax.ShapeDtypeStruct((N_DEV * m_per, n_per), jnp.float32)
    return pl.pallas_call(
        body,
        out_shape=out_shape,
        in_specs=[
            pl.BlockSpec(memory_space=pltpu.VMEM),
            pl.BlockSpec(memory_space=pltpu.VMEM),
        ],
        out_specs=pl.BlockSpec(memory_space=pltpu.VMEM),
        scratch_shapes=[
            pltpu.VMEM((N_SLOTS, half, k), jnp.float32),
            pltpu.VMEM((N_SLOTS, half, k), jnp.float32),
            pltpu.SemaphoreType.DMA((N_SLOTS,)),
            pltpu.SemaphoreType.DMA((N_SLOTS,)),
            pltpu.SemaphoreType.DMA((N_SLOTS,)),
            pltpu.SemaphoreType.DMA((N_SLOTS,)),
            pltpu.SemaphoreType.REGULAR,
            pltpu.SemaphoreType.REGULAR,
        ],
        compiler_params=pltpu.CompilerParams(collective_id=0),
    )(x, w_mat)


# baseline (device time: 338963 ns/iter reference)
import jax
import jax.numpy as jnp
from jax import lax
from jax.experimental import pallas as pl
from jax.experimental.pallas import tpu as pltpu

N_DEV = 8
N_STEPS = N_DEV - 1
N_SLOTS = 4


def kernel(x, w_mat):
    m_per, k = x.shape
    _, n_per = w_mat.shape
    half = m_per // 2

    def body(x_ref, w_ref, out_ref, cw_buf, ccw_buf,
             cw_send_sems, cw_recv_sems, ccw_send_sems, ccw_recv_sems,
             cw_credit, ccw_credit):
        my = lax.axis_index("i")
        right = jnp.mod(my + 1, N_DEV)
        left = jnp.mod(my - 1, N_DEV)

        barrier = pltpu.get_barrier_semaphore()
        pl.semaphore_signal(barrier, inc=1, device_id=(left,),
                            device_id_type=pl.DeviceIdType.MESH)
        pl.semaphore_signal(barrier, inc=1, device_id=(right,),
                            device_id_type=pl.DeviceIdType.MESH)
        pl.semaphore_wait(barrier, 2)

        def desc(s, buf, send_sems, recv_sems, tgt, row0):
            slot = s % N_SLOTS
            if s == 0:
                src = x_ref.at[pl.ds(row0, half), :]
            else:
                src = buf.at[(s - 1) % N_SLOTS]
            return pltpu.make_async_remote_copy(
                src_ref=src,
                dst_ref=buf.at[slot],
                send_sem=send_sems.at[slot],
                recv_sem=recv_sems.at[slot],
                device_id=(tgt,),
                device_id_type=pl.DeviceIdType.MESH,
            )

        def cw_desc(s):
            return desc(s, cw_buf, cw_send_sems, cw_recv_sems, right, 0)

        def ccw_desc(s):
            return desc(s, ccw_buf, ccw_send_sems, ccw_recv_sems, left, half)

        cw_desc(0).start()
        ccw_desc(0).start()

        acc = jnp.dot(x_ref[...], w_ref[...],
                      preferred_element_type=jnp.float32)
        out_ref[pl.ds(my * m_per, m_per), :] = jnp.maximum(acc, 0.0)

        for h in range(N_STEPS):
            cw_desc(h).wait_recv()
            if h + 1 < N_STEPS:
                if h + 1 >= N_SLOTS:
                    pl.semaphore_wait(cw_credit, 1)
                cw_desc(h + 1).start()
            ccw_desc(h).wait_recv()
            if h + 1 < N_STEPS:
                if h + 1 >= N_SLOTS:
                    pl.semaphore_wait(ccw_credit, 1)
                ccw_desc(h + 1).start()

            cw_desc(h).wait_send()
            ccw_desc(h).wait_send()

            if 1 <= h <= N_STEPS - N_SLOTS:
                pl.semaphore_signal(cw_credit, inc=1, device_id=(left,),
                                    device_id_type=pl.DeviceIdType.MESH)
                pl.semaphore_signal(ccw_credit, inc=1, device_id=(right,),
                                    device_id_type=pl.DeviceIdType.MESH)

            o_cw = jnp.mod(my - 1 - h, N_DEV)
            y = jnp.dot(cw_buf[h % N_SLOTS], w_ref[...],
                        preferred_element_type=jnp.float32)
            out_ref[pl.ds(o_cw * m_per, half), :] = jnp.maximum(y, 0.0)

            o_ccw = jnp.mod(my + 1 + h, N_DEV)
            y = jnp.dot(ccw_buf[h % N_SLOTS], w_ref[...],
                        preferred_element_type=jnp.float32)
            out_ref[pl.ds(o_ccw * m_per + half, half), :] = jnp.maximum(y, 0.0)

    out_shape = jax.ShapeDtypeStruct((N_DEV * m_per, n_per), jnp.float32)
    return pl.pallas_call(
        body,
        out_shape=out_shape,
        in_specs=[
            pl.BlockSpec(memory_space=pltpu.VMEM),
            pl.BlockSpec(memory_space=pltpu.VMEM),
        ],
        out_specs=pl.BlockSpec(memory_space=pltpu.VMEM),
        scratch_shapes=[
            pltpu.VMEM((N_SLOTS, half, k), jnp.float32),
            pltpu.VMEM((N_SLOTS, half, k), jnp.float32),
            pltpu.SemaphoreType.DMA((N_SLOTS,)),
            pltpu.SemaphoreType.DMA((N_SLOTS,)),
            pltpu.SemaphoreType.DMA((N_SLOTS,)),
            pltpu.SemaphoreType.DMA((N_SLOTS,)),
            pltpu.SemaphoreType.REGULAR,
            pltpu.SemaphoreType.REGULAR,
        ],
        compiler_params=pltpu.CompilerParams(
            collective_id=0,
            vmem_limit_bytes=46 * 1024 * 1024,
        ),
    )(x, w_mat)


# device time: 240362 ns/iter; 1.4102x vs baseline; 1.4102x over previous
import jax
import jax.numpy as jnp
from jax import lax
from jax.experimental import pallas as pl
from jax.experimental.pallas import tpu as pltpu

N_DEV = 8
N_STEPS = N_DEV - 1
N_SLOTS = 4

B = (
    (1, 2, 1, 4, 1, 2, 1),
    (2, 4, 2, 1, 2, 4, 2),
    (4, 1, 4, 2, 4, 1, 4),
)
PFX = tuple(
    tuple(
        __import__("functools").reduce(lambda a, b: a ^ b, bits[:k])
        for k in range(1, N_STEPS + 1)
    )
    for bits in B
)
ROW0 = (0, 176, 344)
NROWS = (176, 168, 168)


def _pos2v(p):
    return (p & 4) | ((p & 3) ^ ((p >> 1) & 1))


def _v2pos(v):
    return (v & 4) | (v & 2) | ((v ^ (v >> 1)) & 1)


def kernel(x, w_mat):
    m_per, k_dim = x.shape
    _, n_per = w_mat.shape

    def body(x_ref, w_ref, out_ref,
             buf0, buf1, buf2,
             send0, recv0, cred0,
             send1, recv1, cred1,
             send2, recv2, cred2):
        bufs = (buf0, buf1, buf2)
        send_sems = (send0, send1, send2)
        recv_sems = (recv0, recv1, recv2)
        cred_sems = (cred0, cred1, cred2)

        my = lax.axis_index("i")
        myv = _pos2v(my)
        partners = [_v2pos(myv ^ bit) for bit in (1, 2, 4)]

        barrier = pltpu.get_barrier_semaphore()
        for p in partners:
            pl.semaphore_signal(barrier, inc=1, device_id=(p,),
                                device_id_type=pl.DeviceIdType.MESH)
        pl.semaphore_wait(barrier, 3)

        def partner(i, k):
            return _v2pos(myv ^ B[i][k - 1])

        def desc(i, k):
            slot = (k - 1) % N_SLOTS
            if k == 1:
                src = x_ref.at[pl.ds(ROW0[i], NROWS[i]), :]
            else:
                src = bufs[i].at[(k - 2) % N_SLOTS]
            return pltpu.make_async_remote_copy(
                src_ref=src,
                dst_ref=bufs[i].at[slot],
                send_sem=send_sems[i].at[slot],
                recv_sem=recv_sems[i].at[slot],
                device_id=(partner(i, k),),
                device_id_type=pl.DeviceIdType.MESH,
            )

        for i in range(3):
            desc(i, 1).start()

        acc = jnp.dot(x_ref[...], w_ref[...],
                      preferred_element_type=jnp.float32)
        out_ref[pl.ds(my * m_per, m_per), :] = jnp.maximum(acc, 0.0)

        for k in range(1, N_STEPS + 1):
            for i in range(3):
                desc(i, k).wait_recv()
                if k + 1 <= N_STEPS:
                    if k + 1 > N_SLOTS:
                        pl.semaphore_wait(cred_sems[i].at[k % N_SLOTS], 1)
                    desc(i, k + 1).start()

            for i in range(3):
                desc(i, k).wait_send()
                c = k - 1
                if 1 <= c <= N_STEPS - N_SLOTS:
                    pl.semaphore_signal(
                        cred_sems[i].at[(c + 3) % N_SLOTS], inc=1,
                        device_id=(partner(i, c + 4),),
                        device_id_type=pl.DeviceIdType.MESH)

            for i in range(3):
                o = _v2pos(myv ^ PFX[i][k - 1])
                y = jnp.dot(bufs[i][(k - 1) % N_SLOTS], w_ref[...],
                            preferred_element_type=jnp.float32)
                out_ref[pl.ds(o * m_per + ROW0[i], NROWS[i]), :] = (
                    jnp.maximum(y, 0.0))

    out_shape = jax.ShapeDtypeStruct((N_DEV * m_per, n_per), jnp.float32)
    return pl.pallas_call(
        body,
        out_shape=out_shape,
        in_specs=[
            pl.BlockSpec(memory_space=pltpu.VMEM),
            pl.BlockSpec(memory_space=pltpu.VMEM),
        ],
        out_specs=pl.BlockSpec(memory_space=pltpu.VMEM),
        scratch_shapes=[
            pltpu.VMEM((N_SLOTS, NROWS[0], k_dim), jnp.float32),
            pltpu.VMEM((N_SLOTS, NROWS[1], k_dim), jnp.float32),
            pltpu.VMEM((N_SLOTS, NROWS[2], k_dim), jnp.float32),
            pltpu.SemaphoreType.DMA((N_SLOTS,)),
            pltpu.SemaphoreType.DMA((N_SLOTS,)),
            pltpu.SemaphoreType.REGULAR((N_SLOTS,)),
            pltpu.SemaphoreType.DMA((N_SLOTS,)),
            pltpu.SemaphoreType.DMA((N_SLOTS,)),
            pltpu.SemaphoreType.REGULAR((N_SLOTS,)),
            pltpu.SemaphoreType.DMA((N_SLOTS,)),
            pltpu.SemaphoreType.DMA((N_SLOTS,)),
            pltpu.SemaphoreType.REGULAR((N_SLOTS,)),
        ],
        compiler_params=pltpu.CompilerParams(
            collective_id=0,
            vmem_limit_bytes=46 * 1024 * 1024,
        ),
    )(x, w_mat)


# device time: 226985 ns/iter; 1.4933x vs baseline; 1.0589x over previous
import functools

import jax
import jax.numpy as jnp
from jax import lax
from jax.experimental import pallas as pl
from jax.experimental.pallas import tpu as pltpu

N_DEV = 8
N_STEPS = N_DEV - 1
N_SLOTS = 4

B = (
    (1, 2, 1, 4, 1, 2, 1),
    (2, 4, 2, 1, 2, 4, 2),
    (4, 1, 4, 2, 4, 1, 4),
)
PFX = tuple(
    tuple(functools.reduce(lambda a, b: a ^ b, bits[:k])
          for k in range(1, N_STEPS + 1))
    for bits in B
)

PATHS = (
    (0, 0, 88),
    (1, 176, 88),
    (2, 344, 88),
    (0, 88, 88),
    (1, 264, 80),
    (2, 432, 80),
)
N_PATHS = len(PATHS)


def _pos2v(p):
    return (p & 4) | ((p & 3) ^ ((p >> 1) & 1))


def _v2pos(v):
    return (v & 4) | (v & 2) | ((v ^ (v >> 1)) & 1)


def kernel(x, w_mat):
    m_per, k_dim = x.shape
    _, n_per = w_mat.shape

    def body(x_ref, w_ref, out_ref, *scratch):
        bufs = scratch[0::4]
        send_sems = scratch[1::4]
        recv_sems = scratch[2::4]
        cred_sems = scratch[3::4]

        my = lax.axis_index("i")
        myv = _pos2v(my)
        axis_partners = [_v2pos(myv ^ bit) for bit in (1, 2, 4)]

        barrier = pltpu.get_barrier_semaphore()
        for p in axis_partners:
            pl.semaphore_signal(barrier, inc=1, device_id=(p,),
                                device_id_type=pl.DeviceIdType.MESH)
        pl.semaphore_wait(barrier, 3)

        def partner(i, k):
            return _v2pos(myv ^ B[PATHS[i][0]][k - 1])

        def desc(i, k):
            _, row0, nrows = PATHS[i]
            slot = (k - 1) % N_SLOTS
            if k == 1:
                src = x_ref.at[pl.ds(row0, nrows), :]
            else:
                src = bufs[i].at[(k - 2) % N_SLOTS]
            return pltpu.make_async_remote_copy(
                src_ref=src,
                dst_ref=bufs[i].at[slot],
                send_sem=send_sems[i].at[slot],
                recv_sem=recv_sems[i].at[slot],
                device_id=(partner(i, k),),
                device_id_type=pl.DeviceIdType.MESH,
            )

        for i in range(N_PATHS):
            desc(i, 1).start()

        acc = jnp.dot(x_ref[...], w_ref[...],
                      preferred_element_type=jnp.float32)
        out_ref[pl.ds(my * m_per, m_per), :] = jnp.maximum(acc, 0.0)

        for k in range(1, N_STEPS + 1):
            for i in range(N_PATHS):
                desc(i, k).wait_recv()
                if k + 1 <= N_STEPS:
                    if k + 1 > N_SLOTS:
                        pl.semaphore_wait(cred_sems[i].at[k % N_SLOTS], 1)
                    desc(i, k + 1).start()

            for i in range(N_PATHS):
                desc(i, k).wait_send()
                c = k - 1
                if 1 <= c <= N_STEPS - N_SLOTS:
                    pl.semaphore_signal(
                        cred_sems[i].at[(c + 3) % N_SLOTS], inc=1,
                        device_id=(partner(i, c + 4),),
                        device_id_type=pl.DeviceIdType.MESH)

            for i in range(N_PATHS):
                cls, row0, nrows = PATHS[i]
                o = _v2pos(myv ^ PFX[cls][k - 1])
                y = jnp.dot(bufs[i][(k - 1) % N_SLOTS], w_ref[...],
                            preferred_element_type=jnp.float32)
                out_ref[pl.ds(o * m_per + row0, nrows), :] = (
                    jnp.maximum(y, 0.0))

    scratch_shapes = []
    for _, _, nrows in PATHS:
        scratch_shapes += [
            pltpu.VMEM((N_SLOTS, nrows, x.shape[1]), jnp.float32),
            pltpu.SemaphoreType.DMA((N_SLOTS,)),
            pltpu.SemaphoreType.DMA((N_SLOTS,)),
            pltpu.SemaphoreType.REGULAR((N_SLOTS,)),
        ]

    out_shape = jax.ShapeDtypeStruct((N_DEV * m_per, n_per), jnp.float32)
    return pl.pallas_call(
        body,
        out_shape=out_shape,
        in_specs=[
            pl.BlockSpec(memory_space=pltpu.VMEM),
            pl.BlockSpec(memory_space=pltpu.VMEM),
        ],
        out_specs=pl.BlockSpec(memory_space=pltpu.VMEM),
        scratch_shapes=scratch_shapes,
        compiler_params=pltpu.CompilerParams(
            collective_id=0,
            vmem_limit_bytes=46 * 1024 * 1024,
        ),
    )(x, w_mat)


# device time: 225574 ns/iter; 1.5027x vs baseline; 1.0063x over previous
import functools

import jax
import jax.numpy as jnp
from jax import lax
from jax.experimental import pallas as pl
from jax.experimental.pallas import tpu as pltpu

N_DEV = 8
N_STEPS = N_DEV - 1
N_SLOTS = 4

B = (
    (2, 1, 4, 1, 2, 1, 4),
    (4, 2, 1, 2, 4, 2, 1),
    (1, 4, 2, 4, 1, 4, 2),
)
PFX = tuple(
    tuple(functools.reduce(lambda a, b: a ^ b, bits[:k])
          for k in range(1, N_STEPS + 1))
    for bits in B
)

PATHS = (
    (0, 0, 88),
    (1, 176, 88),
    (2, 344, 88),
    (0, 88, 88),
    (1, 264, 80),
    (2, 432, 80),
)
N_PATHS = len(PATHS)


def _pos2v(p):
    return (p & 4) | ((p & 3) ^ ((p >> 1) & 1))


def _v2pos(v):
    return (v & 4) | (v & 2) | ((v ^ (v >> 1)) & 1)


def kernel(x, w_mat):
    m_per, k_dim = x.shape
    _, n_per = w_mat.shape

    def body(x_ref, w_ref, out_ref, *scratch):
        bufs = scratch[0::4]
        send_sems = scratch[1::4]
        recv_sems = scratch[2::4]
        cred_sems = scratch[3::4]

        my = lax.axis_index("i")
        myv = _pos2v(my)
        axis_partners = [_v2pos(myv ^ bit) for bit in (1, 2, 4)]

        barrier = pltpu.get_barrier_semaphore()
        for p in axis_partners:
            pl.semaphore_signal(barrier, inc=1, device_id=(p,),
                                device_id_type=pl.DeviceIdType.MESH)
        pl.semaphore_wait(barrier, 3)

        def partner(i, k):
            return _v2pos(myv ^ B[PATHS[i][0]][k - 1])

        def desc(i, k):
            _, row0, nrows = PATHS[i]
            slot = (k - 1) % N_SLOTS
            if k == 1:
                src = x_ref.at[pl.ds(row0, nrows), :]
            else:
                src = bufs[i].at[(k - 2) % N_SLOTS]
            return pltpu.make_async_remote_copy(
                src_ref=src,
                dst_ref=bufs[i].at[slot],
                send_sem=send_sems[i].at[slot],
                recv_sem=recv_sems[i].at[slot],
                device_id=(partner(i, k),),
                device_id_type=pl.DeviceIdType.MESH,
            )

        for i in range(N_PATHS):
            desc(i, 1).start()

        acc = jnp.dot(x_ref[...], w_ref[...],
                      preferred_element_type=jnp.float32)
        out_ref[pl.ds(my * m_per, m_per), :] = jnp.maximum(acc, 0.0)

        for k in range(1, N_STEPS + 1):
            for i in range(N_PATHS):
                desc(i, k).wait_recv()
                if k + 1 <= N_STEPS:
                    if k + 1 > N_SLOTS:
                        pl.semaphore_wait(cred_sems[i].at[k % N_SLOTS], 1)
                    desc(i, k + 1).start()

            for i in range(N_PATHS):
                desc(i, k).wait_send()
                c = k - 1
                if 1 <= c <= N_STEPS - N_SLOTS:
                    pl.semaphore_signal(
                        cred_sems[i].at[(c + 3) % N_SLOTS], inc=1,
                        device_id=(partner(i, c + 4),),
                        device_id_type=pl.DeviceIdType.MESH)

            for i in range(N_PATHS):
                cls, row0, nrows = PATHS[i]
                o = _v2pos(myv ^ PFX[cls][k - 1])
                y = jnp.dot(bufs[i][(k - 1) % N_SLOTS], w_ref[...],
                            preferred_element_type=jnp.float32)
                out_ref[pl.ds(o * m_per + row0, nrows), :] = (
                    jnp.maximum(y, 0.0))

    scratch_shapes = []
    for _, _, nrows in PATHS:
        scratch_shapes += [
            pltpu.VMEM((N_SLOTS, nrows, x.shape[1]), jnp.float32),
            pltpu.SemaphoreType.DMA((N_SLOTS,)),
            pltpu.SemaphoreType.DMA((N_SLOTS,)),
            pltpu.SemaphoreType.REGULAR((N_SLOTS,)),
        ]

    out_shape = jax.ShapeDtypeStruct((N_DEV * m_per, n_per), jnp.float32)
    return pl.pallas_call(
        body,
        out_shape=out_shape,
        in_specs=[
            pl.BlockSpec(memory_space=pltpu.VMEM),
            pl.BlockSpec(memory_space=pltpu.VMEM),
        ],
        out_specs=pl.BlockSpec(memory_space=pltpu.VMEM),
        scratch_shapes=scratch_shapes,
        compiler_params=pltpu.CompilerParams(
            collective_id=0,
            vmem_limit_bytes=46 * 1024 * 1024,
        ),
    )(x, w_mat)


# device time: 224570 ns/iter; 1.5094x vs baseline; 1.0045x over previous
import functools

import jax
import jax.numpy as jnp
from jax import lax
from jax.experimental import pallas as pl
from jax.experimental.pallas import tpu as pltpu

N_DEV = 8
N_STEPS = N_DEV - 1
N_SLOTS = 4

B = (
    (2, 1, 4, 1, 2, 1, 4),
    (4, 2, 1, 2, 4, 2, 1),
    (1, 4, 2, 4, 1, 4, 2),
)
PFX = tuple(
    tuple(functools.reduce(lambda a, b: a ^ b, bits[:k])
          for k in range(1, N_STEPS + 1))
    for bits in B
)

PATHS = (
    (0, 0, 88),
    (1, 176, 88),
    (2, 344, 88),
    (0, 88, 88),
    (1, 264, 80),
    (2, 432, 80),
)
N_PATHS = len(PATHS)


def _pos2v(p):
    return (p & 4) | ((p & 3) ^ ((p >> 1) & 1))


def _v2pos(v):
    return (v & 4) | (v & 2) | ((v ^ (v >> 1)) & 1)


def kernel(x, w_mat):
    m_per, k_dim = x.shape
    _, n_per = w_mat.shape

    def body(x_ref, w_ref, out_ref, *scratch):
        bufs = scratch[0::4]
        send_sems = scratch[1::4]
        recv_sems = scratch[2::4]
        cred_sems = scratch[3::4]

        my = lax.axis_index("i")
        myv = _pos2v(my)
        axis_partners = [_v2pos(myv ^ bit) for bit in (1, 2, 4)]

        barrier = pltpu.get_barrier_semaphore()
        for p in axis_partners:
            pl.semaphore_signal(barrier, inc=1, device_id=(p,),
                                device_id_type=pl.DeviceIdType.MESH)
        pl.semaphore_wait(barrier, 3)

        def partner(i, k):
            return _v2pos(myv ^ B[PATHS[i][0]][k - 1])

        def desc(i, k):
            _, row0, nrows = PATHS[i]
            slot = (k - 1) % N_SLOTS
            if k == 1:
                src = x_ref.at[pl.ds(row0, nrows), :]
            else:
                src = bufs[i].at[(k - 2) % N_SLOTS]
            return pltpu.make_async_remote_copy(
                src_ref=src,
                dst_ref=bufs[i].at[slot],
                send_sem=send_sems[i].at[slot],
                recv_sem=recv_sems[i].at[slot],
                device_id=(partner(i, k),),
                device_id_type=pl.DeviceIdType.MESH,
            )

        for i in range(N_PATHS):
            desc(i, 1).start()

        acc = jnp.dot(x_ref[...], w_ref[...],
                      preferred_element_type=jnp.float32)
        out_ref[pl.ds(my * m_per, m_per), :] = jnp.maximum(acc, 0.0)

        def gemm_store(i, k):
            cls, row0, nrows = PATHS[i]
            o = _v2pos(myv ^ PFX[cls][k - 1])
            y = jnp.dot(bufs[i][(k - 1) % N_SLOTS], w_ref[...],
                        preferred_element_type=jnp.float32)
            out_ref[pl.ds(o * m_per + row0, nrows), :] = jnp.maximum(y, 0.0)

        for k in range(1, N_STEPS + 1):
            for i in range(N_PATHS):
                desc(i, k).wait_recv()
                if k + 1 <= N_STEPS:
                    if k + 1 > N_SLOTS:
                        pl.semaphore_wait(cred_sems[i].at[k % N_SLOTS], 1)
                    desc(i, k + 1).start()
                else:
                    gemm_store(i, k)

            for i in range(N_PATHS):
                desc(i, k).wait_send()
                c = k - 1
                if 1 <= c <= N_STEPS - N_SLOTS:
                    pl.semaphore_signal(
                        cred_sems[i].at[(c + 3) % N_SLOTS], inc=1,
                        device_id=(partner(i, c + 4),),
                        device_id_type=pl.DeviceIdType.MESH)

            if k < N_STEPS:
                for i in range(N_PATHS):
                    gemm_store(i, k)

    scratch_shapes = []
    for _, _, nrows in PATHS:
        scratch_shapes += [
            pltpu.VMEM((N_SLOTS, nrows, x.shape[1]), jnp.float32),
            pltpu.SemaphoreType.DMA((N_SLOTS,)),
            pltpu.SemaphoreType.DMA((N_SLOTS,)),
            pltpu.SemaphoreType.REGULAR((N_SLOTS,)),
        ]

    out_shape = jax.ShapeDtypeStruct((N_DEV * m_per, n_per), jnp.float32)
    return pl.pallas_call(
        body,
        out_shape=out_shape,
        in_specs=[
            pl.BlockSpec(memory_space=pltpu.VMEM),
            pl.BlockSpec(memory_space=pltpu.VMEM),
        ],
        out_specs=pl.BlockSpec(memory_space=pltpu.VMEM),
        scratch_shapes=scratch_shapes,
        compiler_params=pltpu.CompilerParams(
            collective_id=0,
            vmem_limit_bytes=46 * 1024 * 1024,
        ),
    )(x, w_mat)
